# baseline (speedup 1.0000x reference)
import functools

import jax
import jax.numpy as jnp
from jax import lax
from jax.experimental import pallas as pl
from jax.experimental.pallas import tpu as pltpu
from jax.experimental.pallas import tpu_sc as plsc

_N = 10000
_E = 320000
_DIN = 128
_DH = 256
_EPS = 1e-5

_NPAD = 10240
_B = 128
_NW = 32
_EB = 10240
_EPAD = _EB * _NW
_NBLK = _EB // _B
_TROWS = _NPAD // 16
_SB = 32
_NSTG = _EPAD // (_B * _SB)
_S0 = _NSTG // 16

_mesh = plsc.VectorSubcoreMesh(core_axis_name="c", subcore_axis_name="s")


@functools.partial(
    pl.kernel,
    out_type=[jax.ShapeDtypeStruct((_NPAD,), jnp.float32),
              jax.ShapeDtypeStruct((_NPAD,), jnp.float32)],
    mesh=_mesh,
    scratch_types=[
        pltpu.VMEM((_NBLK, _B), jnp.int32),
        pltpu.VMEM((_B,), jnp.float32),
        pltpu.VMEM((_TROWS,), jnp.float32),
        pltpu.VMEM_SHARED((_NPAD,), jnp.float32),
    ],
)
def _deg_sc(dst_hbm, out0, out1, dst_v, ones_v, z_v, deg_sp):
    c = lax.axis_index("c")
    s = lax.axis_index("s")
    wid = s * 2 + c
    for j in range(_B // 16):
        ones_v[pl.ds(j * 16, 16)] = jnp.ones((16,), jnp.float32)
    for j in range(_TROWS // 16):
        z_v[pl.ds(j * 16, 16)] = jnp.zeros((16,), jnp.float32)
    pltpu.sync_copy(dst_hbm.at[pl.ds(wid * _NBLK, _NBLK)], dst_v)
    pltpu.sync_copy(z_v, deg_sp.at[pl.ds(s * _TROWS, _TROWS)])
    plsc.subcore_barrier()

    def body(j, carry):
        pltpu.sync_copy(ones_v, deg_sp.at[dst_v.at[j]], add=True)
        return carry

    lax.fori_loop(0, _NBLK, body, 0)
    plsc.subcore_barrier()

    @pl.when(c == 0)
    def _():
        pltpu.sync_copy(deg_sp.at[pl.ds(s * _TROWS, _TROWS)],
                        out0.at[pl.ds(s * _TROWS, _TROWS)])

    @pl.when(c == 1)
    def _():
        pltpu.sync_copy(deg_sp.at[pl.ds(s * _TROWS, _TROWS)],
                        out1.at[pl.ds(s * _TROWS, _TROWS)])


@functools.partial(
    pl.kernel,
    out_type=jax.ShapeDtypeStruct((_NPAD, 128), jnp.float32),
    mesh=_mesh,
    scratch_types=[
        pltpu.VMEM((_SB, _B), jnp.int32),
        pltpu.VMEM((_SB, _B), jnp.int32),
        pltpu.VMEM((_B, 128), jnp.float32),
        pltpu.VMEM((_B, 128), jnp.float32),
        pltpu.VMEM_SHARED((_NPAD, 128), jnp.float32),
        pltpu.SemaphoreType.DMA,
    ],
)
def _prop_sc(table_hbm, src_hbm, dst_hbm, out0,
             src_v, dst_v, rows0_v, rows1_v, acc_sp, sem):
    c = lax.axis_index("c")
    s = lax.axis_index("s")

    @pl.when(c == 0)
    def _():
        for i in range(_B):
            for j in range(8):
                rows0_v[i, pl.ds(j * 16, 16)] = jnp.zeros((16,), jnp.float32)
        for k in range(_TROWS // _B):
            pltpu.sync_copy(rows0_v, acc_sp.at[pl.ds(s * _TROWS + k * _B, _B)])

    plsc.subcore_barrier()

    def gather(j, buf):
        return pltpu.make_async_copy(table_hbm.at[src_v.at[j]], buf, sem)

    @pl.when(c == 0)
    def _():
        for st in range(_S0):
            srow = (s * _S0 + st) * _SB
            pltpu.sync_copy(src_hbm.at[pl.ds(srow, _SB)], src_v)
            pltpu.sync_copy(dst_hbm.at[pl.ds(srow, _SB)], dst_v)
            gather(0, rows0_v).start()

            def body(k, carry):
                j0 = k * 2
                j1 = j0 + 1
                gather(j0, rows0_v).wait()
                gather(j1, rows1_v).start()
                pltpu.sync_copy(rows0_v, acc_sp.at[dst_v.at[j0]], add=True)
                gather(j1, rows1_v).wait()

                @pl.when(k < _SB // 2 - 1)
                def _():
                    gather(j0 + 2, rows0_v).start()

                pltpu.sync_copy(rows1_v, acc_sp.at[dst_v.at[j1]], add=True)
                return carry

            lax.fori_loop(0, _SB // 2, body, 0)

    plsc.subcore_barrier()

    @pl.when(c == 0)
    def _():
        pltpu.sync_copy(acc_sp.at[pl.ds(s * _TROWS, _TROWS)],
                        out0.at[pl.ds(s * _TROWS, _TROWS)])


_S2 = _S0


@functools.partial(
    pl.kernel,
    out_type=[jax.ShapeDtypeStruct((_NPAD, 128), jnp.float32),
              jax.ShapeDtypeStruct((_NPAD, 128), jnp.float32)],
    mesh=_mesh,
    scratch_types=[
        pltpu.VMEM((_SB, _B), jnp.int32),
        pltpu.VMEM((_SB, _B), jnp.int32),
        pltpu.VMEM((_B, 128), jnp.float32),
        pltpu.VMEM((_B, 128), jnp.float32),
        pltpu.VMEM_SHARED((_NPAD, 128), jnp.float32),
        pltpu.SemaphoreType.DMA,
    ],
)
def _prop2_sc(tlo_hbm, thi_hbm, src_hbm, dst_hbm, out_lo, out_hi,
              src_v, dst_v, rows0_v, rows1_v, acc_sp, sem):
    c = lax.axis_index("c")
    s = lax.axis_index("s")
    for i in range(_B):
        for j in range(8):
            rows0_v[i, pl.ds(j * 16, 16)] = jnp.zeros((16,), jnp.float32)
    for k in range(_TROWS // _B):
        pltpu.sync_copy(rows0_v, acc_sp.at[pl.ds(s * _TROWS + k * _B, _B)])
    plsc.subcore_barrier()

    def run(table_hbm):
        def gather(j, buf):
            return pltpu.make_async_copy(table_hbm.at[src_v.at[j]], buf, sem)

        for st in range(_S2):
            srow = (s * _S2 + st) * _SB
            pltpu.sync_copy(src_hbm.at[pl.ds(srow, _SB)], src_v)
            pltpu.sync_copy(dst_hbm.at[pl.ds(srow, _SB)], dst_v)
            gather(0, rows0_v).start()

            def body(k, carry):
                j0 = k * 2
                j1 = j0 + 1
                gather(j0, rows0_v).wait()
                gather(j1, rows1_v).start()
                pltpu.sync_copy(rows0_v, acc_sp.at[dst_v.at[j0]], add=True)
                gather(j1, rows1_v).wait()

                @pl.when(k < _SB // 2 - 1)
                def _():
                    gather(j0 + 2, rows0_v).start()

                pltpu.sync_copy(rows1_v, acc_sp.at[dst_v.at[j1]], add=True)
                return carry

            lax.fori_loop(0, _SB // 2, body, 0)

    @pl.when(c == 0)
    def _():
        run(tlo_hbm)

    @pl.when(c == 1)
    def _():
        run(thi_hbm)

    plsc.subcore_barrier()

    @pl.when(c == 0)
    def _():
        pltpu.sync_copy(acc_sp.at[pl.ds(s * _TROWS, _TROWS)],
                        out_lo.at[pl.ds(s * _TROWS, _TROWS)])

    @pl.when(c == 1)
    def _():
        pltpu.sync_copy(acc_sp.at[pl.ds(s * _TROWS, _TROWS)],
                        out_hi.at[pl.ds(s * _TROWS, _TROWS)])


_BLK = 1000
_G = _N // _BLK


def _row_spec(d):
    return pl.BlockSpec((_BLK, d), lambda i: (i, 0))


def _full_spec(r, d):
    return pl.BlockSpec((r, d), lambda i: (0, 0))


def _stats_spec(d):
    return pl.BlockSpec((2, d), lambda i: (0, 0))


def _acc_stats(i, st_ref, r):
    blk = jnp.concatenate([jnp.sum(r, 0, keepdims=True),
                           jnp.sum(r * r, 0, keepdims=True)], 0)

    @pl.when(i == 0)
    def _():
        st_ref[...] = blk

    @pl.when(i > 0)
    def _():
        st_ref[...] = st_ref[...] + blk


def _bn(r, st, g, b):
    mu = st[0:1] / _N
    var = st[1:2] / _N - mu * mu
    return g * (r - mu) * lax.rsqrt(var + _EPS) + b


def _k_xstats(x_ref, st_ref):
    _acc_stats(pl.program_id(0), st_ref, x_ref[...])


def _k_prep(x_ref, st_ref, g_ref, b_ref, d0_ref, d1_ref, xbp_ref, dinv_ref):
    xn = _bn(x_ref[...], st_ref[...], g_ref[...], b_ref[...])
    dv = lax.rsqrt(d0_ref[...] + d1_ref[...] + 1.0)
    dinv_ref[...] = dv
    xbp_ref[...] = dv * xn


def _k_layer1(p0_ref, xbp_ref, dinv_ref, w_ref, b_ref, r_ref, st_ref):
    sprop = dinv_ref[...] * (p0_ref[...] + xbp_ref[...])
    h = jnp.dot(sprop, w_ref[...], preferred_element_type=jnp.float32) + b_ref[...]
    r = jnp.maximum(h, 0.0)
    r_ref[...] = r
    _acc_stats(pl.program_id(0), st_ref, r)


def _k_norm_mm(with_res):
    def body(*refs):
        if with_res:
            (r_ref, st_ref, g_ref, b_ref, xprev_ref, w_ref, dinv_ref,
             xp_ref, hlo_ref, hhi_ref) = refs
        else:
            (r_ref, st_ref, g_ref, b_ref, w_ref, dinv_ref,
             xp_ref, hlo_ref, hhi_ref) = refs
        x = _bn(r_ref[...], st_ref[...], g_ref[...], b_ref[...])
        if with_res:
            x = x + xprev_ref[...]
        xp_ref[...] = x
        hp = dinv_ref[...] * jnp.dot(x, w_ref[...],
                                     preferred_element_type=jnp.float32)
        hlo_ref[...] = hp[:, :128]
        hhi_ref[...] = hp[:, 128:]
    return body


def _k_combine(plo, phi, hlo, hhi, dinv_ref, b_ref, r_ref, st_ref):
    t = jnp.concatenate([plo[...] + hlo[...],
                         phi[...] + hhi[...]], axis=1)
    t = dinv_ref[...] * t + b_ref[...]
    r = jnp.maximum(t, 0.0)
    r_ref[...] = r
    _acc_stats(pl.program_id(0), st_ref, r)


def _k_out(r_ref, st_ref, g_ref, b_ref, xprev_ref, w_ref, bo_ref, o_ref):
    x = _bn(r_ref[...], st_ref[...], g_ref[...], b_ref[...])
    xp = x + xprev_ref[...]
    o_ref[...] = jnp.dot(xp, w_ref[...],
                         preferred_element_type=jnp.float32) + bo_ref[...]


def _pc(body, in_specs, out_shapes, out_specs):
    return pl.pallas_call(
        body, grid=(_G,), in_specs=in_specs,
        out_shape=out_shapes, out_specs=out_specs)


def kernel(x, edge_index, bn_in_g, bn_in_b, W1, b1, W2, b2, W3, b3,
           bn1_g, bn1_b, bn2_g, bn2_b, bn3_g, bn3_b, W_out, b_out):
    f32 = jnp.float32
    src = edge_index[0]
    dst = edge_index[1]
    npad = _EPAD - _E
    srcp = jnp.concatenate([src, jnp.zeros((npad,), jnp.int32)]
                           ).reshape(_EPAD // _B, _B)
    dstp = jnp.concatenate([dst, jnp.full((npad,), _N, jnp.int32)]
                           ).reshape(_EPAD // _B, _B)

    g_in = bn_in_g.reshape(1, _DIN)
    b_in = bn_in_b.reshape(1, _DIN)
    b1r = b1.reshape(1, _DH)
    b2r = b2.reshape(1, _DH)
    b3r = b3.reshape(1, _DH)
    g1 = bn1_g.reshape(1, _DH)
    be1 = bn1_b.reshape(1, _DH)
    g2 = bn2_g.reshape(1, _DH)
    be2 = bn2_b.reshape(1, _DH)
    g3 = bn3_g.reshape(1, _DH)
    be3 = bn3_b.reshape(1, _DH)
    w_out_pad = jnp.zeros((_DH, 128), f32).at[:, :2].set(W_out)
    b_out_pad = jnp.zeros((1, 128), f32).at[:, :2].set(b_out.reshape(1, 2))

    d0, d1 = _deg_sc(dstp)
    d0c = d0.reshape(_NPAD, 1)
    d1c = d1.reshape(_NPAD, 1)

    st_x = _pc(_k_xstats, [_row_spec(_DIN)],
               jax.ShapeDtypeStruct((2, _DIN), f32), _stats_spec(_DIN))(x)

    xbp, dinv = _pc(
        _k_prep,
        [_row_spec(_DIN), _stats_spec(_DIN), _full_spec(1, _DIN),
         _full_spec(1, _DIN), _row_spec(1), _row_spec(1)],
        [jax.ShapeDtypeStruct((_N, _DIN), f32),
         jax.ShapeDtypeStruct((_N, 1), f32)],
        [_row_spec(_DIN), _row_spec(1)],
    )(x, st_x, g_in, b_in, d0c, d1c)

    p0 = _prop_sc(xbp, srcp, dstp)
    r1, st1 = _pc(
        _k_layer1,
        [_row_spec(128), _row_spec(_DIN), _row_spec(1),
         _full_spec(_DIN, _DH), _full_spec(1, _DH)],
        [jax.ShapeDtypeStruct((_N, _DH), f32),
         jax.ShapeDtypeStruct((2, _DH), f32)],
        [_row_spec(_DH), _stats_spec(_DH)],
    )(p0, xbp, dinv, W1, b1r)

    def norm_mm(r, st, g, be, W, xprev):
        with_res = xprev is not None
        specs = [_row_spec(_DH), _stats_spec(_DH), _full_spec(1, _DH),
                 _full_spec(1, _DH)]
        args = [r, st, g, be]
        if with_res:
            specs.append(_row_spec(_DH))
            args.append(xprev)
        specs += [_full_spec(_DH, _DH), _row_spec(1)]
        args += [W, dinv]
        return _pc(
            _k_norm_mm(with_res), specs,
            [jax.ShapeDtypeStruct((_N, _DH), f32),
             jax.ShapeDtypeStruct((_N, 128), f32),
             jax.ShapeDtypeStruct((_N, 128), f32)],
            [_row_spec(_DH), _row_spec(128), _row_spec(128)],
        )(*args)

    def combine(plo, phi, hlo, hhi, br):
        return _pc(
            _k_combine,
            [_row_spec(128)] * 4 + [_row_spec(1), _full_spec(1, _DH)],
            [jax.ShapeDtypeStruct((_N, _DH), f32),
             jax.ShapeDtypeStruct((2, _DH), f32)],
            [_row_spec(_DH), _stats_spec(_DH)],
        )(plo, phi, hlo, hhi, dinv, br)

    xp1, hlo2, hhi2 = norm_mm(r1, st1, g1, be1, W2, None)
    plo, phi = _prop2_sc(hlo2, hhi2, srcp, dstp)
    r2, st2 = combine(plo, phi, hlo2, hhi2, b2r)

    xp2, hlo3, hhi3 = norm_mm(r2, st2, g2, be2, W3, xp1)
    plo, phi = _prop2_sc(hlo3, hhi3, srcp, dstp)
    r3, st3 = combine(plo, phi, hlo3, hhi3, b3r)

    o = _pc(
        _k_out,
        [_row_spec(_DH), _stats_spec(_DH), _full_spec(1, _DH),
         _full_spec(1, _DH), _row_spec(_DH), _full_spec(_DH, 128),
         _full_spec(1, 128)],
        jax.ShapeDtypeStruct((_N, 128), f32),
        _row_spec(128),
    )(r3, st3, g3, be3, xp2, w_out_pad, b_out_pad)
    return o[:, :2]


# speedup vs baseline: 1.0351x; 1.0351x over previous
import functools

import jax
import jax.numpy as jnp
from jax import lax
from jax.experimental import pallas as pl
from jax.experimental.pallas import tpu as pltpu
from jax.experimental.pallas import tpu_sc as plsc

_N = 10000
_E = 320000
_DIN = 128
_DH = 256
_EPS = 1e-5

_NPAD = 10240
_B = 128
_NW = 32
_EB = 10240
_EPAD = _EB * _NW
_NBLK = _EB // _B
_TROWS = _NPAD // 16
_SB = 32
_NSTG = _EPAD // (_B * _SB)
_S0 = 4
_S1 = 1

_mesh = plsc.VectorSubcoreMesh(core_axis_name="c", subcore_axis_name="s")


@functools.partial(
    pl.kernel,
    out_type=[jax.ShapeDtypeStruct((_NPAD,), jnp.float32),
              jax.ShapeDtypeStruct((_NPAD,), jnp.float32)],
    mesh=_mesh,
    scratch_types=[
        pltpu.VMEM((_NBLK, _B), jnp.int32),
        pltpu.VMEM((_B,), jnp.float32),
        pltpu.VMEM((_TROWS,), jnp.float32),
        pltpu.VMEM_SHARED((_NPAD,), jnp.float32),
    ],
)
def _deg_sc(dst_hbm, out0, out1, dst_v, ones_v, z_v, deg_sp):
    c = lax.axis_index("c")
    s = lax.axis_index("s")
    wid = s * 2 + c
    for j in range(_B // 16):
        ones_v[pl.ds(j * 16, 16)] = jnp.ones((16,), jnp.float32)
    for j in range(_TROWS // 16):
        z_v[pl.ds(j * 16, 16)] = jnp.zeros((16,), jnp.float32)
    pltpu.sync_copy(dst_hbm.at[pl.ds(wid * _NBLK, _NBLK)], dst_v)
    pltpu.sync_copy(z_v, deg_sp.at[pl.ds(s * _TROWS, _TROWS)])
    plsc.subcore_barrier()

    def body(j, carry):
        pltpu.sync_copy(ones_v, deg_sp.at[dst_v.at[j]], add=True)
        return carry

    lax.fori_loop(0, _NBLK, body, 0)
    plsc.subcore_barrier()

    @pl.when(c == 0)
    def _():
        pltpu.sync_copy(deg_sp.at[pl.ds(s * _TROWS, _TROWS)],
                        out0.at[pl.ds(s * _TROWS, _TROWS)])

    @pl.when(c == 1)
    def _():
        pltpu.sync_copy(deg_sp.at[pl.ds(s * _TROWS, _TROWS)],
                        out1.at[pl.ds(s * _TROWS, _TROWS)])


@functools.partial(
    pl.kernel,
    out_type=[jax.ShapeDtypeStruct((_NPAD, 128), jnp.float32),
              jax.ShapeDtypeStruct((_NPAD, 128), jnp.float32)],
    mesh=_mesh,
    scratch_types=[
        pltpu.VMEM((_SB, _B), jnp.int32),
        pltpu.VMEM((_SB, _B), jnp.int32),
        pltpu.VMEM((_B, 128), jnp.float32),
        pltpu.VMEM((_B, 128), jnp.float32),
        pltpu.VMEM_SHARED((_NPAD, 128), jnp.float32),
        pltpu.SemaphoreType.DMA,
    ],
)
def _prop_sc(table_hbm, src_hbm, dst_hbm, out0, out1,
             src_v, dst_v, rows0_v, rows1_v, acc_sp, sem):
    c = lax.axis_index("c")
    s = lax.axis_index("s")
    for i in range(_B):
        for j in range(8):
            rows0_v[i, pl.ds(j * 16, 16)] = jnp.zeros((16,), jnp.float32)
    for k in range(_TROWS // _B):
        pltpu.sync_copy(rows0_v, acc_sp.at[pl.ds(s * _TROWS + k * _B, _B)])

    plsc.subcore_barrier()

    def gather(j, buf):
        return pltpu.make_async_copy(table_hbm.at[src_v.at[j]], buf, sem)

    def run(stage0, nstages):
        for st in range(nstages):
            srow = (stage0 + st) * _SB
            pltpu.sync_copy(src_hbm.at[pl.ds(srow, _SB)], src_v)
            pltpu.sync_copy(dst_hbm.at[pl.ds(srow, _SB)], dst_v)
            gather(0, rows0_v).start()

            def body(k, carry):
                j0 = k * 2
                j1 = j0 + 1
                gather(j0, rows0_v).wait()
                gather(j1, rows1_v).start()
                pltpu.sync_copy(rows0_v, acc_sp.at[dst_v.at[j0]], add=True)
                gather(j1, rows1_v).wait()

                @pl.when(k < _SB // 2 - 1)
                def _():
                    gather(j0 + 2, rows0_v).start()

                pltpu.sync_copy(rows1_v, acc_sp.at[dst_v.at[j1]], add=True)
                return carry

            lax.fori_loop(0, _SB // 2, body, 0)

    @pl.when(c == 0)
    def _():
        run(s * _S0, _S0)

    @pl.when(c == 1)
    def _():
        run(16 * _S0 + s * _S1, _S1)

    plsc.subcore_barrier()

    @pl.when(c == 0)
    def _():
        pltpu.sync_copy(acc_sp.at[pl.ds(s * _TROWS, _TROWS)],
                        out0.at[pl.ds(s * _TROWS, _TROWS)])

    @pl.when(c == 1)
    def _():
        pltpu.sync_copy(acc_sp.at[pl.ds(s * _TROWS, _TROWS)],
                        out1.at[pl.ds(s * _TROWS, _TROWS)])


_S2 = _NSTG // 16


@functools.partial(
    pl.kernel,
    out_type=[jax.ShapeDtypeStruct((_NPAD, 128), jnp.float32),
              jax.ShapeDtypeStruct((_NPAD, 128), jnp.float32)],
    mesh=_mesh,
    scratch_types=[
        pltpu.VMEM((_SB, _B), jnp.int32),
        pltpu.VMEM((_SB, _B), jnp.int32),
        pltpu.VMEM((_B, 128), jnp.float32),
        pltpu.VMEM((_B, 128), jnp.float32),
        pltpu.VMEM_SHARED((_NPAD, 128), jnp.float32),
        pltpu.SemaphoreType.DMA,
    ],
)
def _prop2_sc(tlo_hbm, thi_hbm, src_hbm, dst_hbm, out_lo, out_hi,
              src_v, dst_v, rows0_v, rows1_v, acc_sp, sem):
    c = lax.axis_index("c")
    s = lax.axis_index("s")
    for i in range(_B):
        for j in range(8):
            rows0_v[i, pl.ds(j * 16, 16)] = jnp.zeros((16,), jnp.float32)
    for k in range(_TROWS // _B):
        pltpu.sync_copy(rows0_v, acc_sp.at[pl.ds(s * _TROWS + k * _B, _B)])
    plsc.subcore_barrier()

    def run(table_hbm):
        def gather(j, buf):
            return pltpu.make_async_copy(table_hbm.at[src_v.at[j]], buf, sem)

        for st in range(_S2):
            srow = (s * _S2 + st) * _SB
            pltpu.sync_copy(src_hbm.at[pl.ds(srow, _SB)], src_v)
            pltpu.sync_copy(dst_hbm.at[pl.ds(srow, _SB)], dst_v)
            gather(0, rows0_v).start()

            def body(k, carry):
                j0 = k * 2
                j1 = j0 + 1
                gather(j0, rows0_v).wait()
                gather(j1, rows1_v).start()
                pltpu.sync_copy(rows0_v, acc_sp.at[dst_v.at[j0]], add=True)
                gather(j1, rows1_v).wait()

                @pl.when(k < _SB // 2 - 1)
                def _():
                    gather(j0 + 2, rows0_v).start()

                pltpu.sync_copy(rows1_v, acc_sp.at[dst_v.at[j1]], add=True)
                return carry

            lax.fori_loop(0, _SB // 2, body, 0)

    @pl.when(c == 0)
    def _():
        run(tlo_hbm)

    @pl.when(c == 1)
    def _():
        run(thi_hbm)

    plsc.subcore_barrier()

    @pl.when(c == 0)
    def _():
        pltpu.sync_copy(acc_sp.at[pl.ds(s * _TROWS, _TROWS)],
                        out_lo.at[pl.ds(s * _TROWS, _TROWS)])

    @pl.when(c == 1)
    def _():
        pltpu.sync_copy(acc_sp.at[pl.ds(s * _TROWS, _TROWS)],
                        out_hi.at[pl.ds(s * _TROWS, _TROWS)])


_BLK = 1000
_G = _N // _BLK


def _row_spec(d):
    return pl.BlockSpec((_BLK, d), lambda i: (i, 0))


def _full_spec(r, d):
    return pl.BlockSpec((r, d), lambda i: (0, 0))


def _stats_spec(d):
    return pl.BlockSpec((2, d), lambda i: (0, 0))


def _acc_stats(i, st_ref, r):
    blk = jnp.concatenate([jnp.sum(r, 0, keepdims=True),
                           jnp.sum(r * r, 0, keepdims=True)], 0)

    @pl.when(i == 0)
    def _():
        st_ref[...] = blk

    @pl.when(i > 0)
    def _():
        st_ref[...] = st_ref[...] + blk


def _bn(r, st, g, b):
    mu = st[0:1] / _N
    var = st[1:2] / _N - mu * mu
    return g * (r - mu) * lax.rsqrt(var + _EPS) + b


def _k_xstats(x_ref, st_ref):
    _acc_stats(pl.program_id(0), st_ref, x_ref[...])


def _k_prep(x_ref, st_ref, g_ref, b_ref, d0_ref, d1_ref, xbp_ref, dinv_ref):
    xn = _bn(x_ref[...], st_ref[...], g_ref[...], b_ref[...])
    dv = lax.rsqrt(d0_ref[...] + d1_ref[...] + 1.0)
    dinv_ref[...] = dv
    xbp_ref[...] = dv * xn


def _k_layer1(p0_ref, p1_ref, xbp_ref, dinv_ref, w_ref, b_ref, r_ref, st_ref):
    sprop = dinv_ref[...] * (p0_ref[...] + p1_ref[...] + xbp_ref[...])
    h = jnp.dot(sprop, w_ref[...], preferred_element_type=jnp.float32) + b_ref[...]
    r = jnp.maximum(h, 0.0)
    r_ref[...] = r
    _acc_stats(pl.program_id(0), st_ref, r)


def _k_norm_mm(with_res):
    def body(*refs):
        if with_res:
            (r_ref, st_ref, g_ref, b_ref, xprev_ref, w_ref, dinv_ref,
             xp_ref, hlo_ref, hhi_ref) = refs
        else:
            (r_ref, st_ref, g_ref, b_ref, w_ref, dinv_ref,
             xp_ref, hlo_ref, hhi_ref) = refs
        x = _bn(r_ref[...], st_ref[...], g_ref[...], b_ref[...])
        if with_res:
            x = x + xprev_ref[...]
        xp_ref[...] = x
        hp = dinv_ref[...] * jnp.dot(x, w_ref[...],
                                     preferred_element_type=jnp.float32)
        hlo_ref[...] = hp[:, :128]
        hhi_ref[...] = hp[:, 128:]
    return body


def _k_combine(plo, phi, hlo, hhi, dinv_ref, b_ref, r_ref, st_ref):
    t = jnp.concatenate([plo[...] + hlo[...],
                         phi[...] + hhi[...]], axis=1)
    t = dinv_ref[...] * t + b_ref[...]
    r = jnp.maximum(t, 0.0)
    r_ref[...] = r
    _acc_stats(pl.program_id(0), st_ref, r)


def _k_out(r_ref, st_ref, g_ref, b_ref, xprev_ref, w_ref, bo_ref, o_ref):
    x = _bn(r_ref[...], st_ref[...], g_ref[...], b_ref[...])
    xp = x + xprev_ref[...]
    o_ref[...] = jnp.dot(xp, w_ref[...],
                         preferred_element_type=jnp.float32) + bo_ref[...]


def _pc(body, in_specs, out_shapes, out_specs):
    return pl.pallas_call(
        body, grid=(_G,), in_specs=in_specs,
        out_shape=out_shapes, out_specs=out_specs)


def kernel(x, edge_index, bn_in_g, bn_in_b, W1, b1, W2, b2, W3, b3,
           bn1_g, bn1_b, bn2_g, bn2_b, bn3_g, bn3_b, W_out, b_out):
    f32 = jnp.float32
    src = edge_index[0]
    dst = edge_index[1]
    npad = _EPAD - _E
    srcp = jnp.concatenate([src, jnp.zeros((npad,), jnp.int32)]
                           ).reshape(_EPAD // _B, _B)
    dstp = jnp.concatenate([dst, jnp.full((npad,), _N, jnp.int32)]
                           ).reshape(_EPAD // _B, _B)

    g_in = bn_in_g.reshape(1, _DIN)
    b_in = bn_in_b.reshape(1, _DIN)
    b1r = b1.reshape(1, _DH)
    b2r = b2.reshape(1, _DH)
    b3r = b3.reshape(1, _DH)
    g1 = bn1_g.reshape(1, _DH)
    be1 = bn1_b.reshape(1, _DH)
    g2 = bn2_g.reshape(1, _DH)
    be2 = bn2_b.reshape(1, _DH)
    g3 = bn3_g.reshape(1, _DH)
    be3 = bn3_b.reshape(1, _DH)
    w_out_pad = jnp.zeros((_DH, 128), f32).at[:, :2].set(W_out)
    b_out_pad = jnp.zeros((1, 128), f32).at[:, :2].set(b_out.reshape(1, 2))

    d0, d1 = _deg_sc(dstp)
    d0c = d0.reshape(_NPAD, 1)
    d1c = d1.reshape(_NPAD, 1)

    st_x = _pc(_k_xstats, [_row_spec(_DIN)],
               jax.ShapeDtypeStruct((2, _DIN), f32), _stats_spec(_DIN))(x)

    xbp, dinv = _pc(
        _k_prep,
        [_row_spec(_DIN), _stats_spec(_DIN), _full_spec(1, _DIN),
         _full_spec(1, _DIN), _row_spec(1), _row_spec(1)],
        [jax.ShapeDtypeStruct((_N, _DIN), f32),
         jax.ShapeDtypeStruct((_N, 1), f32)],
        [_row_spec(_DIN), _row_spec(1)],
    )(x, st_x, g_in, b_in, d0c, d1c)

    p0, p1 = _prop_sc(xbp, srcp, dstp)
    r1, st1 = _pc(
        _k_layer1,
        [_row_spec(128), _row_spec(128), _row_spec(_DIN), _row_spec(1),
         _full_spec(_DIN, _DH), _full_spec(1, _DH)],
        [jax.ShapeDtypeStruct((_N, _DH), f32),
         jax.ShapeDtypeStruct((2, _DH), f32)],
        [_row_spec(_DH), _stats_spec(_DH)],
    )(p0, p1, xbp, dinv, W1, b1r)

    def norm_mm(r, st, g, be, W, xprev):
        with_res = xprev is not None
        specs = [_row_spec(_DH), _stats_spec(_DH), _full_spec(1, _DH),
                 _full_spec(1, _DH)]
        args = [r, st, g, be]
        if with_res:
            specs.append(_row_spec(_DH))
            args.append(xprev)
        specs += [_full_spec(_DH, _DH), _row_spec(1)]
        args += [W, dinv]
        return _pc(
            _k_norm_mm(with_res), specs,
            [jax.ShapeDtypeStruct((_N, _DH), f32),
             jax.ShapeDtypeStruct((_N, 128), f32),
             jax.ShapeDtypeStruct((_N, 128), f32)],
            [_row_spec(_DH), _row_spec(128), _row_spec(128)],
        )(*args)

    def combine(plo, phi, hlo, hhi, br):
        return _pc(
            _k_combine,
            [_row_spec(128)] * 4 + [_row_spec(1), _full_spec(1, _DH)],
            [jax.ShapeDtypeStruct((_N, _DH), f32),
             jax.ShapeDtypeStruct((2, _DH), f32)],
            [_row_spec(_DH), _stats_spec(_DH)],
        )(plo, phi, hlo, hhi, dinv, br)

    xp1, hlo2, hhi2 = norm_mm(r1, st1, g1, be1, W2, None)
    plo, phi = _prop2_sc(hlo2, hhi2, srcp, dstp)
    r2, st2 = combine(plo, phi, hlo2, hhi2, b2r)

    xp2, hlo3, hhi3 = norm_mm(r2, st2, g2, be2, W3, xp1)
    plo, phi = _prop2_sc(hlo3, hhi3, srcp, dstp)
    r3, st3 = combine(plo, phi, hlo3, hhi3, b3r)

    o = _pc(
        _k_out,
        [_row_spec(_DH), _stats_spec(_DH), _full_spec(1, _DH),
         _full_spec(1, _DH), _row_spec(_DH), _full_spec(_DH, 128),
         _full_spec(1, 128)],
        jax.ShapeDtypeStruct((_N, 128), f32),
        _row_spec(128),
    )(r3, st3, g3, be3, xp2, w_out_pad, b_out_pad)
    return o[:, :2]
